# baseline (speedup 1.0000x reference)
import dataclasses

import jax
import jax.numpy as jnp
from jax import lax
from jax.experimental import pallas as pl
from jax.experimental.pallas import tpu as pltpu
from jax.experimental.pallas import tpu_sc as plsc

_D = 33
_PLANE = _D * _D * _D
_PLANE_PAD = 35952

_B, _C, _H, _W = 4, 3, 512, 512
_NWORKERS = 32
_BAND = _H // 8
_NCHUNK = (_BAND // 8) * (_W // 128)


def _coords(v):
    s = v * 2.0 - 1.0
    f = (s + 1.0) * (0.5 * (_D - 1))
    f = jnp.minimum(jnp.maximum(f, 0.0), float(_D - 1))
    i0 = f.astype(jnp.int32)
    w = f - i0.astype(jnp.float32)
    i1 = jnp.minimum(i0 + 1, _D - 1)
    return i0, i1, w


def _compute(lut_r, lut_g, lut_b, rb, gb, bb, orb, ogb, obb):
    @plsc.parallel_loop(0, 64, 1, unroll=2)
    def _vec(j):
        sl = (j // 8, pl.ds((j % 8) * 16, 16))
        x0, x1, wx = _coords(rb[sl])
        y0, y1, wy = _coords(gb[sl])
        z0, z1, wz = _coords(bb[sl])

        ya = y0 * _D
        yb = y1 * _D
        za = z0 * (_D * _D)
        zb = z1 * (_D * _D)
        i000 = za + ya + x0
        i001 = za + ya + x1
        i010 = za + yb + x0
        i011 = za + yb + x1
        i100 = zb + ya + x0
        i101 = zb + ya + x1
        i110 = zb + yb + x0
        i111 = zb + yb + x1

        ex = 1.0 - wx
        ey = 1.0 - wy
        ez = 1.0 - wz
        a00 = ey * ez
        a10 = wy * ez
        a01 = ey * wz
        a11 = wy * wz
        w000 = ex * a00
        w001 = wx * a00
        w010 = ex * a10
        w011 = wx * a10
        w100 = ex * a01
        w101 = wx * a01
        w110 = ex * a11
        w111 = wx * a11

        for lv, ob in ((lut_r, orb), (lut_g, ogb), (lut_b, obb)):
            acc = w000 * plsc.load_gather(lv, [i000])
            acc += w001 * plsc.load_gather(lv, [i001])
            acc += w010 * plsc.load_gather(lv, [i010])
            acc += w011 * plsc.load_gather(lv, [i011])
            acc += w100 * plsc.load_gather(lv, [i100])
            acc += w101 * plsc.load_gather(lv, [i101])
            acc += w110 * plsc.load_gather(lv, [i110])
            acc += w111 * plsc.load_gather(lv, [i111])
            ob[sl] = acc


def _tile_slice(img, ci, band):
    r0 = band * _BAND + (ci // (_W // 128)) * 8
    c0 = (ci % (_W // 128)) * 128
    return img, pl.ds(r0, 8), pl.ds(c0, 128)


def _in_copies(x_hbm, img, ci, band, bufs, sem):
    b, rs, cs = _tile_slice(img, ci, band)
    return [
        pltpu.make_async_copy(x_hbm.at[b, k, rs, cs], bufs[k], sem)
        for k in range(3)
    ]


def _out_copies(o_hbm, img, ci, band, bufs, sem):
    b, rs, cs = _tile_slice(img, ci, band)
    return [
        pltpu.make_async_copy(bufs[k], o_hbm.at[b, k, rs, cs], sem)
        for k in range(3)
    ]


def _body(lut_hbm, x_hbm, o_hbm, lut_r, lut_g, lut_b,
          ra, ga, ba, rb2, gb2, bb2,
          ora, oga, oba, orb, ogb, obb,
          isem_a, isem_b, osem_a, osem_b):
    wid = lax.axis_index("c") * 16 + lax.axis_index("s")
    pltpu.sync_copy(lut_hbm.at[pl.ds(0, _PLANE_PAD)], lut_r)
    pltpu.sync_copy(lut_hbm.at[pl.ds(_PLANE_PAD, _PLANE_PAD)], lut_g)
    pltpu.sync_copy(lut_hbm.at[pl.ds(2 * _PLANE_PAD, _PLANE_PAD)], lut_b)

    img = wid // 8
    band = wid % 8
    in_a, in_b = (ra, ga, ba), (rb2, gb2, bb2)
    out_a, out_b = (ora, oga, oba), (orb, ogb, obb)

    for c in _in_copies(x_hbm, img, 0, band, in_a, isem_a):
        c.start()

    @pl.loop(0, _NCHUNK, step=2)
    def _chunk(ci):
        @pl.when(ci > 0)
        def _():
            for c in _out_copies(o_hbm, img, ci - 2, band, out_a, osem_a):
                c.wait()
        for c in _in_copies(x_hbm, img, ci, band, in_a, isem_a):
            c.wait()
        for c in _in_copies(x_hbm, img, ci + 1, band, in_b, isem_b):
            c.start()
        _compute(lut_r, lut_g, lut_b, *in_a, *out_a)
        for c in _out_copies(o_hbm, img, ci, band, out_a, osem_a):
            c.start()

        @pl.when(ci > 0)
        def _():
            for c in _out_copies(o_hbm, img, ci - 1, band, out_b, osem_b):
                c.wait()
        for c in _in_copies(x_hbm, img, ci + 1, band, in_b, isem_b):
            c.wait()

        @pl.when(ci + 2 < _NCHUNK)
        def _():
            for c in _in_copies(x_hbm, img, ci + 2, band, in_a, isem_a):
                c.start()
        _compute(lut_r, lut_g, lut_b, *in_b, *out_b)
        for c in _out_copies(o_hbm, img, ci + 1, band, out_b, osem_b):
            c.start()

    for c in _out_copies(o_hbm, img, _NCHUNK - 2, band, out_a, osem_a):
        c.wait()
    for c in _out_copies(o_hbm, img, _NCHUNK - 1, band, out_b, osem_b):
        c.wait()


def kernel(lut, x):
    lut_pad = jnp.pad(
        lut.reshape(3, _PLANE), ((0, 0), (0, _PLANE_PAD - _PLANE))
    ).reshape(-1)

    cp = pltpu.CompilerParams()
    if "needs_layout_passes" in pltpu.CompilerParams.__dataclass_fields__:
        cp = dataclasses.replace(cp, needs_layout_passes=False)
    cp = dataclasses.replace(cp, use_tc_tiling_on_sc=True)

    mesh = plsc.VectorSubcoreMesh(core_axis_name="c", subcore_axis_name="s")
    buf = pltpu.VMEM((8, 128), jnp.float32)
    fn = pl.kernel(
        _body,
        out_type=jax.ShapeDtypeStruct((_B, _C, _H, _W), jnp.float32),
        mesh=mesh,
        scratch_types=[pltpu.VMEM((_PLANE_PAD,), jnp.float32)] * 3
        + [buf] * 12
        + [pltpu.SemaphoreType.DMA] * 4,
        compiler_params=cp,
    )
    return fn(lut_pad, x)


# speedup vs baseline: 1.1948x; 1.1948x over previous
import dataclasses

import jax
import jax.numpy as jnp
from jax import lax
from jax.experimental import pallas as pl
from jax.experimental.pallas import tpu as pltpu
from jax.experimental.pallas import tpu_sc as plsc

_D = 33
_PLANE = _D * _D * _D
_LUT_PAD = 107824

_B, _C, _H, _W = 4, 3, 512, 512
_NWORKERS = 32
_BAND = _H // 8
_NCHUNK = (_BAND // 8) * (_W // 128)


def _coords(v):
    s = v * 2.0 - 1.0
    f = (s + 1.0) * (0.5 * (_D - 1))
    f = jnp.minimum(jnp.maximum(f, 0.0), float(_D - 1))
    i0 = f.astype(jnp.int32)
    w = f - i0.astype(jnp.float32)
    i1 = jnp.minimum(i0 + 1, _D - 1)
    return i0, i1, w


def _compute(lut_v, rb, gb, bb, orb, ogb, obb):
    @plsc.parallel_loop(0, 64, 1, unroll=2)
    def _vec(j):
        sl = (j // 8, pl.ds((j % 8) * 16, 16))
        x0, x1, wx = _coords(rb[sl])
        y0, y1, wy = _coords(gb[sl])
        z0, z1, wz = _coords(bb[sl])

        ya = y0 * _D
        yb = y1 * _D
        za = z0 * (_D * _D)
        zb = z1 * (_D * _D)
        i000 = za + ya + x0
        i001 = za + ya + x1
        i010 = za + yb + x0
        i011 = za + yb + x1
        i100 = zb + ya + x0
        i101 = zb + ya + x1
        i110 = zb + yb + x0
        i111 = zb + yb + x1

        ex = 1.0 - wx
        ey = 1.0 - wy
        ez = 1.0 - wz
        a00 = ey * ez
        a10 = wy * ez
        a01 = ey * wz
        a11 = wy * wz
        w000 = ex * a00
        w001 = wx * a00
        w010 = ex * a10
        w011 = wx * a10
        w100 = ex * a01
        w101 = wx * a01
        w110 = ex * a11
        w111 = wx * a11

        for c, ob in ((0, orb), (1, ogb), (2, obb)):
            oc = c * _PLANE
            acc = w000 * plsc.load_gather(lut_v, [i000 + oc])
            acc += w001 * plsc.load_gather(lut_v, [i001 + oc])
            acc += w010 * plsc.load_gather(lut_v, [i010 + oc])
            acc += w011 * plsc.load_gather(lut_v, [i011 + oc])
            acc += w100 * plsc.load_gather(lut_v, [i100 + oc])
            acc += w101 * plsc.load_gather(lut_v, [i101 + oc])
            acc += w110 * plsc.load_gather(lut_v, [i110 + oc])
            acc += w111 * plsc.load_gather(lut_v, [i111 + oc])
            ob[sl] = acc


def _tile_slice(img, ci, band):
    r0 = band * _BAND + (ci // (_W // 128)) * 8
    c0 = (ci % (_W // 128)) * 128
    return img, pl.ds(r0, 8), pl.ds(c0, 128)


def _in_copies(x_hbm, img, ci, band, bufs, sem):
    b, rs, cs = _tile_slice(img, ci, band)
    return [
        pltpu.make_async_copy(x_hbm.at[b, k, rs, cs], bufs[k], sem)
        for k in range(3)
    ]


def _out_copies(o_hbm, img, ci, band, bufs, sem):
    b, rs, cs = _tile_slice(img, ci, band)
    return [
        pltpu.make_async_copy(bufs[k], o_hbm.at[b, k, rs, cs], sem)
        for k in range(3)
    ]


def _body(lut_hbm, x_hbm, o_hbm, lut_v,
          ra, ga, ba, rb2, gb2, bb2,
          ora, oga, oba, orb, ogb, obb,
          isem_a, isem_b, osem_a, osem_b):
    wid = lax.axis_index("c") * 16 + lax.axis_index("s")
    pltpu.sync_copy(lut_hbm, lut_v)

    img = wid // 8
    band = wid % 8
    in_a, in_b = (ra, ga, ba), (rb2, gb2, bb2)
    out_a, out_b = (ora, oga, oba), (orb, ogb, obb)

    for c in _in_copies(x_hbm, img, 0, band, in_a, isem_a):
        c.start()

    @pl.loop(0, _NCHUNK, step=2)
    def _chunk(ci):
        @pl.when(ci > 0)
        def _():
            for c in _out_copies(o_hbm, img, ci - 2, band, out_a, osem_a):
                c.wait()
        for c in _in_copies(x_hbm, img, ci, band, in_a, isem_a):
            c.wait()
        for c in _in_copies(x_hbm, img, ci + 1, band, in_b, isem_b):
            c.start()
        _compute(lut_v, *in_a, *out_a)
        for c in _out_copies(o_hbm, img, ci, band, out_a, osem_a):
            c.start()

        @pl.when(ci > 0)
        def _():
            for c in _out_copies(o_hbm, img, ci - 1, band, out_b, osem_b):
                c.wait()
        for c in _in_copies(x_hbm, img, ci + 1, band, in_b, isem_b):
            c.wait()

        @pl.when(ci + 2 < _NCHUNK)
        def _():
            for c in _in_copies(x_hbm, img, ci + 2, band, in_a, isem_a):
                c.start()
        _compute(lut_v, *in_b, *out_b)
        for c in _out_copies(o_hbm, img, ci + 1, band, out_b, osem_b):
            c.start()

    for c in _out_copies(o_hbm, img, _NCHUNK - 2, band, out_a, osem_a):
        c.wait()
    for c in _out_copies(o_hbm, img, _NCHUNK - 1, band, out_b, osem_b):
        c.wait()


def kernel(lut, x):
    lut_pad = jnp.pad(lut.reshape(-1), (0, _LUT_PAD - 3 * _PLANE))

    cp = pltpu.CompilerParams()
    if "needs_layout_passes" in pltpu.CompilerParams.__dataclass_fields__:
        cp = dataclasses.replace(cp, needs_layout_passes=False)
    cp = dataclasses.replace(cp, use_tc_tiling_on_sc=True)

    mesh = plsc.VectorSubcoreMesh(core_axis_name="c", subcore_axis_name="s")
    buf = pltpu.VMEM((8, 128), jnp.float32)
    fn = pl.kernel(
        _body,
        out_type=jax.ShapeDtypeStruct((_B, _C, _H, _W), jnp.float32),
        mesh=mesh,
        scratch_types=[pltpu.VMEM((_LUT_PAD,), jnp.float32)]
        + [buf] * 12
        + [pltpu.SemaphoreType.DMA] * 4,
        compiler_params=cp,
    )
    return fn(lut_pad, x)


# speedup vs baseline: 1.2221x; 1.0228x over previous
import dataclasses

import jax
import jax.numpy as jnp
from jax import lax
from jax.experimental import pallas as pl
from jax.experimental.pallas import tpu as pltpu
from jax.experimental.pallas import tpu_sc as plsc

_D = 33
_PLANE = _D * _D * _D
_LUT_PAD = 107824

_B, _C, _H, _W = 4, 3, 512, 512
_NWORKERS = 32
_BAND = _H // 8
_NCHUNK = (_BAND // 8) * (_W // 128)


def _coords(v):
    s = v * 2.0 - 1.0
    f = (s + 1.0) * (0.5 * (_D - 1))
    f = jnp.minimum(jnp.maximum(f, 0.0), float(_D - 1))
    i0 = f.astype(jnp.int32)
    w = f - i0.astype(jnp.float32)
    i1 = jnp.minimum(i0 + 1, _D - 1)
    return i0, i1, w


def _compute(lut_v, rb, gb, bb, orb, ogb, obb):
    @plsc.parallel_loop(0, 64, 1, unroll=3)
    def _vec(j):
        sl = (j // 8, pl.ds((j % 8) * 16, 16))
        x0, x1, wx = _coords(rb[sl])
        y0, y1, wy = _coords(gb[sl])
        z0, z1, wz = _coords(bb[sl])

        ya = y0 * _D
        yb = y1 * _D
        za = z0 * (_D * _D)
        zb = z1 * (_D * _D)
        i000 = za + ya + x0
        i001 = za + ya + x1
        i010 = za + yb + x0
        i011 = za + yb + x1
        i100 = zb + ya + x0
        i101 = zb + ya + x1
        i110 = zb + yb + x0
        i111 = zb + yb + x1

        ex = 1.0 - wx
        ey = 1.0 - wy
        ez = 1.0 - wz
        a00 = ey * ez
        a10 = wy * ez
        a01 = ey * wz
        a11 = wy * wz
        w000 = ex * a00
        w001 = wx * a00
        w010 = ex * a10
        w011 = wx * a10
        w100 = ex * a01
        w101 = wx * a01
        w110 = ex * a11
        w111 = wx * a11

        for c, ob in ((0, orb), (1, ogb), (2, obb)):
            oc = c * _PLANE
            acc = w000 * plsc.load_gather(lut_v, [i000 + oc])
            acc += w001 * plsc.load_gather(lut_v, [i001 + oc])
            acc += w010 * plsc.load_gather(lut_v, [i010 + oc])
            acc += w011 * plsc.load_gather(lut_v, [i011 + oc])
            acc += w100 * plsc.load_gather(lut_v, [i100 + oc])
            acc += w101 * plsc.load_gather(lut_v, [i101 + oc])
            acc += w110 * plsc.load_gather(lut_v, [i110 + oc])
            acc += w111 * plsc.load_gather(lut_v, [i111 + oc])
            ob[sl] = acc


def _tile_slice(img, ci, band):
    r0 = band * _BAND + (ci // (_W // 128)) * 8
    c0 = (ci % (_W // 128)) * 128
    return img, pl.ds(r0, 8), pl.ds(c0, 128)


def _in_copies(x_hbm, img, ci, band, bufs, sem):
    b, rs, cs = _tile_slice(img, ci, band)
    return [
        pltpu.make_async_copy(x_hbm.at[b, k, rs, cs], bufs[k], sem)
        for k in range(3)
    ]


def _out_copies(o_hbm, img, ci, band, bufs, sem):
    b, rs, cs = _tile_slice(img, ci, band)
    return [
        pltpu.make_async_copy(bufs[k], o_hbm.at[b, k, rs, cs], sem)
        for k in range(3)
    ]


def _body(lut_hbm, x_hbm, o_hbm, lut_v,
          ra, ga, ba, rb2, gb2, bb2,
          ora, oga, oba, orb, ogb, obb,
          isem_a, isem_b, osem_a, osem_b):
    wid = lax.axis_index("c") * 16 + lax.axis_index("s")
    pltpu.sync_copy(lut_hbm, lut_v)

    img = wid // 8
    band = wid % 8
    in_a, in_b = (ra, ga, ba), (rb2, gb2, bb2)
    out_a, out_b = (ora, oga, oba), (orb, ogb, obb)

    for c in _in_copies(x_hbm, img, 0, band, in_a, isem_a):
        c.start()

    @pl.loop(0, _NCHUNK, step=2)
    def _chunk(ci):
        @pl.when(ci > 0)
        def _():
            for c in _out_copies(o_hbm, img, ci - 2, band, out_a, osem_a):
                c.wait()
        for c in _in_copies(x_hbm, img, ci, band, in_a, isem_a):
            c.wait()
        for c in _in_copies(x_hbm, img, ci + 1, band, in_b, isem_b):
            c.start()
        _compute(lut_v, *in_a, *out_a)
        for c in _out_copies(o_hbm, img, ci, band, out_a, osem_a):
            c.start()

        @pl.when(ci > 0)
        def _():
            for c in _out_copies(o_hbm, img, ci - 1, band, out_b, osem_b):
                c.wait()
        for c in _in_copies(x_hbm, img, ci + 1, band, in_b, isem_b):
            c.wait()

        @pl.when(ci + 2 < _NCHUNK)
        def _():
            for c in _in_copies(x_hbm, img, ci + 2, band, in_a, isem_a):
                c.start()
        _compute(lut_v, *in_b, *out_b)
        for c in _out_copies(o_hbm, img, ci + 1, band, out_b, osem_b):
            c.start()

    for c in _out_copies(o_hbm, img, _NCHUNK - 2, band, out_a, osem_a):
        c.wait()
    for c in _out_copies(o_hbm, img, _NCHUNK - 1, band, out_b, osem_b):
        c.wait()


def kernel(lut, x):
    lut_pad = jnp.pad(lut.reshape(-1), (0, _LUT_PAD - 3 * _PLANE))

    cp = pltpu.CompilerParams()
    if "needs_layout_passes" in pltpu.CompilerParams.__dataclass_fields__:
        cp = dataclasses.replace(cp, needs_layout_passes=False)
    cp = dataclasses.replace(cp, use_tc_tiling_on_sc=True)

    mesh = plsc.VectorSubcoreMesh(core_axis_name="c", subcore_axis_name="s")
    buf = pltpu.VMEM((8, 128), jnp.float32)
    fn = pl.kernel(
        _body,
        out_type=jax.ShapeDtypeStruct((_B, _C, _H, _W), jnp.float32),
        mesh=mesh,
        scratch_types=[pltpu.VMEM((_LUT_PAD,), jnp.float32)]
        + [buf] * 12
        + [pltpu.SemaphoreType.DMA] * 4,
        compiler_params=cp,
    )
    return fn(lut_pad, x)
